# Initial kernel scaffold; baseline (speedup 1.0000x reference)
import jax
import jax.numpy as jnp
from jax.experimental import pallas as pl


def kernel(x, codebook):
    raise NotImplementedError("write your pallas kernel here")


# speedup vs baseline: 1.0308x; 1.0308x over previous
import functools

import jax
import jax.numpy as jnp
from jax import lax
from jax.experimental import pallas as pl
from jax.experimental.pallas import tpu as pltpu
from jax.experimental.pallas import tpu_sc as plsc

_BN = 256
_BK = 2048


def _argmin_body(x_ref, cbt_ref, idx_ref, loss_ref, csq_ref, *, bk, nsteps, scale):
    i = pl.program_id(0)

    @pl.when(i == 0)
    def _init():
        cbt = cbt_ref[...]
        csq_ref[...] = jnp.sum(cbt * cbt, axis=0, keepdims=True)
        loss_ref[0, 0] = 0.0

    xb = x_ref[...]
    bn = xb.shape[0]
    xsq = jnp.sum(xb * xb, axis=1, keepdims=True)
    k_total = nsteps * bk

    def chunk_min(j, carry):
        best, bidx = carry
        cbt_c = cbt_ref[:, pl.ds(j * bk, bk)]
        s = lax.dot_general(
            xb, cbt_c, (((1,), (0,)), ((), ())),
            preferred_element_type=jnp.float32,
            precision=lax.Precision.DEFAULT,
        )
        e = (xsq + csq_ref[:, pl.ds(j * bk, bk)]) - 2.0 * s
        m = jnp.min(e, axis=1, keepdims=True)
        iota = lax.broadcasted_iota(jnp.int32, (bn, bk), 1)
        li = jnp.min(jnp.where(e == m, iota, k_total), axis=1, keepdims=True) + j * bk
        upd = m < best
        return jnp.where(upd, m, best), jnp.where(upd, li, bidx)

    inf0 = jnp.full((bn, 1), jnp.inf, jnp.float32)
    zi = jnp.zeros((bn, 1), jnp.int32)
    best, bidx = lax.fori_loop(0, nsteps, chunk_min, (inf0, zi))

    idx_ref[0, 0, :] = bidx[:, 0]
    loss_ref[0, 0] += jnp.sum(best) * scale


def _argmin_call(x, cbt):
    n, d = x.shape
    k = cbt.shape[1]
    nblk = n // _BN
    idx3, loss = pl.pallas_call(
        functools.partial(_argmin_body, bk=_BK, nsteps=k // _BK,
                          scale=1.0 / (n * d)),
        grid=(nblk,),
        in_specs=[
            pl.BlockSpec((_BN, d), lambda i: (i, 0)),
            pl.BlockSpec((d, k), lambda i: (0, 0)),
        ],
        out_specs=[
            pl.BlockSpec((1, 1, _BN), lambda i: (i, 0, 0)),
            pl.BlockSpec(memory_space=pltpu.SMEM, block_shape=(1, 1),
                         index_map=lambda i: (0, 0)),
        ],
        out_shape=[
            jax.ShapeDtypeStruct((nblk, 1, _BN), jnp.int32),
            jax.ShapeDtypeStruct((1, 1), jnp.float32),
        ],
        scratch_shapes=[pltpu.VMEM((1, k), jnp.float32)],
        compiler_params=pltpu.CompilerParams(
            dimension_semantics=("arbitrary",)),
    )(x, cbt)
    return idx3.reshape(n), loss[0, 0]


@functools.lru_cache(maxsize=None)
def _gather_make(v, d, b, ch):
    info = plsc.get_sparse_core_info()
    nw = info.num_cores * info.num_subcores
    b_per_w = b // nw
    nch = b_per_w // ch
    mesh = plsc.VectorSubcoreMesh(core_axis_name="c", subcore_axis_name="s")

    @functools.partial(
        pl.kernel, mesh=mesh,
        out_type=jax.ShapeDtypeStruct((b, d), jnp.float32),
        compiler_params=pltpu.CompilerParams(use_tc_tiling_on_sc=False),
        scratch_types=[
            pltpu.VMEM((ch,), jnp.int32),
            pltpu.VMEM((ch, d), jnp.float32),
            pltpu.SemaphoreType.DMA,
        ],
    )
    def gk(table_hbm, idx_hbm, out_hbm, idx_v, rows_v, sem):
        wid = lax.axis_index("s") * info.num_cores + lax.axis_index("c")
        base = wid * b_per_w
        for c in range(nch):
            o = base + c * ch
            pltpu.sync_copy(idx_hbm.at[pl.ds(o, ch)], idx_v)
            pltpu.async_copy(table_hbm.at[idx_v], rows_v, sem).wait()
            pltpu.sync_copy(rows_v, out_hbm.at[pl.ds(o, ch)])

    return gk


def kernel(x, codebook):
    n, d = x.shape
    indices, loss = _argmin_call(x, codebook.T)
    quantized_st = _gather_make(codebook.shape[0], d, n, 1024)(codebook, indices)
    return (quantized_st, indices, loss)


# speedup vs baseline: 1.4167x; 1.3744x over previous
import functools

import jax
import jax.numpy as jnp
from jax import lax
from jax.experimental import pallas as pl
from jax.experimental.pallas import tpu as pltpu
from jax.experimental.pallas import tpu_sc as plsc

_BN = 512
_BK = 2048


def _argmin_body(x_ref, cbt_ref, idx_ref, loss_ref, csq_ref, *, bk, nsteps, scale):
    i = pl.program_id(0)

    @pl.when(i == 0)
    def _init():
        cbt = cbt_ref[...]
        csq_ref[...] = jnp.sum(cbt * cbt, axis=0, keepdims=True)
        loss_ref[0, 0] = 0.0

    xb = x_ref[...]
    bn = xb.shape[0]
    xsq = jnp.sum(xb * xb, axis=1, keepdims=True)
    xb2 = -2.0 * xb
    k_total = nsteps * bk

    def chunk_min(j, carry):
        best, bidx = carry
        cbt_c = cbt_ref[:, pl.ds(j * bk, bk)]
        s2 = lax.dot_general(
            xb2, cbt_c, (((1,), (0,)), ((), ())),
            preferred_element_type=jnp.float32,
            precision=lax.Precision.DEFAULT,
        )
        e = (xsq + csq_ref[:, pl.ds(j * bk, bk)]) + s2
        m = jnp.min(e, axis=1, keepdims=True)
        iota = lax.broadcasted_iota(jnp.int32, (bn, bk), 1)
        li = jnp.min(jnp.where(e == m, iota, k_total), axis=1, keepdims=True) + j * bk
        upd = m < best
        return jnp.where(upd, m, best), jnp.where(upd, li, bidx)

    inf0 = jnp.full((bn, 1), jnp.inf, jnp.float32)
    zi = jnp.zeros((bn, 1), jnp.int32)
    carry = (inf0, zi)
    for j in range(nsteps):
        carry = chunk_min(j, carry)
    best, bidx = carry

    idx_ref[0, 0, :] = bidx[:, 0]
    loss_ref[0, 0] += jnp.sum(best) * scale


def _argmin_call(x, cbt):
    n, d = x.shape
    k = cbt.shape[1]
    nblk = n // _BN
    idx3, loss = pl.pallas_call(
        functools.partial(_argmin_body, bk=_BK, nsteps=k // _BK,
                          scale=1.0 / (n * d)),
        grid=(nblk,),
        in_specs=[
            pl.BlockSpec((_BN, d), lambda i: (i, 0)),
            pl.BlockSpec((d, k), lambda i: (0, 0)),
        ],
        out_specs=[
            pl.BlockSpec((1, 1, _BN), lambda i: (i, 0, 0)),
            pl.BlockSpec(memory_space=pltpu.SMEM, block_shape=(1, 1),
                         index_map=lambda i: (0, 0)),
        ],
        out_shape=[
            jax.ShapeDtypeStruct((nblk, 1, _BN), jnp.int32),
            jax.ShapeDtypeStruct((1, 1), jnp.float32),
        ],
        scratch_shapes=[pltpu.VMEM((1, k), jnp.float32)],
        compiler_params=pltpu.CompilerParams(
            dimension_semantics=("arbitrary",)),
    )(x, cbt)
    return idx3.reshape(n), loss[0, 0]


@functools.lru_cache(maxsize=None)
def _gather_make(v, d, b, ch):
    info = plsc.get_sparse_core_info()
    nw = info.num_cores * info.num_subcores
    b_per_w = b // nw
    nch = b_per_w // ch
    mesh = plsc.VectorSubcoreMesh(core_axis_name="c", subcore_axis_name="s")

    @functools.partial(
        pl.kernel, mesh=mesh,
        out_type=jax.ShapeDtypeStruct((b, d), jnp.float32),
        compiler_params=pltpu.CompilerParams(use_tc_tiling_on_sc=False),
        scratch_types=[
            pltpu.VMEM((ch,), jnp.int32),
            pltpu.VMEM((ch, d), jnp.float32),
            pltpu.SemaphoreType.DMA,
        ],
    )
    def gk(table_hbm, idx_hbm, out_hbm, idx_v, rows_v, sem):
        wid = lax.axis_index("s") * info.num_cores + lax.axis_index("c")
        base = wid * b_per_w
        for c in range(nch):
            o = base + c * ch
            pltpu.sync_copy(idx_hbm.at[pl.ds(o, ch)], idx_v)
            pltpu.async_copy(table_hbm.at[idx_v], rows_v, sem).wait()
            pltpu.sync_copy(rows_v, out_hbm.at[pl.ds(o, ch)])

    return gk


def kernel(x, codebook):
    n, d = x.shape
    indices, loss = _argmin_call(x, codebook.T)
    quantized_st = _gather_make(codebook.shape[0], d, n, 1024)(codebook, indices)
    return (quantized_st, indices, loss)


# speedup vs baseline: 1.6145x; 1.1396x over previous
import functools

import jax
import jax.numpy as jnp
from jax import lax
from jax.experimental import pallas as pl
from jax.experimental.pallas import tpu as pltpu
from jax.experimental.pallas import tpu_sc as plsc

_BN = 512
_BK = 8192


def _argmin_body(x_ref, cbt_ref, idx_ref, loss_ref, csq_ref, *, bk, nsteps, scale):
    i = pl.program_id(0)

    @pl.when(i == 0)
    def _init():
        cbt = cbt_ref[...]
        csq_ref[...] = jnp.sum(cbt * cbt, axis=0, keepdims=True)
        loss_ref[0, 0] = 0.0

    xb = x_ref[...]
    bn = xb.shape[0]
    xsq = jnp.sum(xb * xb, axis=1, keepdims=True)
    xb2 = -2.0 * xb
    k_total = nsteps * bk

    def chunk_min(j, carry):
        best, bidx = carry
        cbt_c = cbt_ref[:, pl.ds(j * bk, bk)]
        s2 = lax.dot_general(
            xb2, cbt_c, (((1,), (0,)), ((), ())),
            preferred_element_type=jnp.float32,
            precision=lax.Precision.DEFAULT,
        )
        e = (xsq + csq_ref[:, pl.ds(j * bk, bk)]) + s2
        m = jnp.min(e, axis=1, keepdims=True)
        li = jnp.argmin(e, axis=1).astype(jnp.int32)[:, None] + j * bk
        upd = m < best
        return jnp.where(upd, m, best), jnp.where(upd, li, bidx)

    inf0 = jnp.full((bn, 1), jnp.inf, jnp.float32)
    zi = jnp.zeros((bn, 1), jnp.int32)
    carry = (inf0, zi)
    for j in range(nsteps):
        carry = chunk_min(j, carry)
    best, bidx = carry

    idx_ref[0, 0, :] = bidx[:, 0]
    loss_ref[0, 0] += jnp.sum(best) * scale


def _argmin_call(x, cbt):
    n, d = x.shape
    k = cbt.shape[1]
    nblk = n // _BN
    idx3, loss = pl.pallas_call(
        functools.partial(_argmin_body, bk=_BK, nsteps=k // _BK,
                          scale=1.0 / (n * d)),
        grid=(nblk,),
        in_specs=[
            pl.BlockSpec((_BN, d), lambda i: (i, 0)),
            pl.BlockSpec((d, k), lambda i: (0, 0)),
        ],
        out_specs=[
            pl.BlockSpec((1, 1, _BN), lambda i: (i, 0, 0)),
            pl.BlockSpec(memory_space=pltpu.SMEM, block_shape=(1, 1),
                         index_map=lambda i: (0, 0)),
        ],
        out_shape=[
            jax.ShapeDtypeStruct((nblk, 1, _BN), jnp.int32),
            jax.ShapeDtypeStruct((1, 1), jnp.float32),
        ],
        scratch_shapes=[pltpu.VMEM((1, k), jnp.float32)],
        compiler_params=pltpu.CompilerParams(
            dimension_semantics=("arbitrary",)),
    )(x, cbt)
    return idx3.reshape(n), loss[0, 0]


@functools.lru_cache(maxsize=None)
def _gather_make(v, d, b, ch):
    info = plsc.get_sparse_core_info()
    nw = info.num_cores * info.num_subcores
    b_per_w = b // nw
    nch = b_per_w // ch
    mesh = plsc.VectorSubcoreMesh(core_axis_name="c", subcore_axis_name="s")

    @functools.partial(
        pl.kernel, mesh=mesh,
        out_type=jax.ShapeDtypeStruct((b, d), jnp.float32),
        compiler_params=pltpu.CompilerParams(use_tc_tiling_on_sc=False),
        scratch_types=[
            pltpu.VMEM((ch,), jnp.int32),
            pltpu.VMEM((ch, d), jnp.float32),
            pltpu.SemaphoreType.DMA,
        ],
    )
    def gk(table_hbm, idx_hbm, out_hbm, idx_v, rows_v, sem):
        wid = lax.axis_index("s") * info.num_cores + lax.axis_index("c")
        base = wid * b_per_w
        for c in range(nch):
            o = base + c * ch
            pltpu.sync_copy(idx_hbm.at[pl.ds(o, ch)], idx_v)
            pltpu.async_copy(table_hbm.at[idx_v], rows_v, sem).wait()
            pltpu.sync_copy(rows_v, out_hbm.at[pl.ds(o, ch)])

    return gk


def kernel(x, codebook):
    n, d = x.shape
    indices, loss = _argmin_call(x, codebook.T)
    quantized_st = _gather_make(codebook.shape[0], d, n, 1024)(codebook, indices)
    return (quantized_st, indices, loss)
